# Initial kernel scaffold; baseline (speedup 1.0000x reference)
import jax
import jax.numpy as jnp
from jax import lax
from jax.experimental import pallas as pl
from jax.experimental.pallas import tpu as pltpu
from jax.experimental.pallas import tpu_sc as plsc

B = 16384
D_TITLE, D_AUTHOR, D_PUB, D_YEAR_PAD = 32, 32, 16, 16
HIDDEN = 512
EMBED_DIM = 128
K_PAD = D_TITLE + D_AUTHOR + D_PUB + D_YEAR_PAD

NC, NS = 2, 16
NW = NC * NS
B_PER_W = B // NW
CHUNK = 128
N_CHUNKS = B_PER_W // CHUNK


def _gather_body(it_h, ia_h, ip_h, iy_h, tt_h, ta_h, tp_h, ty_h,
                 et_h, ea_h, ep_h, ey_h,
                 iv_t, iv_a, iv_p, iv_y, rt, ra, rp, ry, sem):
    wid = lax.axis_index("s") * NC + lax.axis_index("c")
    base = wid * B_PER_W
    pltpu.sync_copy(it_h.at[wid], iv_t)
    pltpu.sync_copy(ia_h.at[wid], iv_a)
    pltpu.sync_copy(ip_h.at[wid], iv_p)
    pltpu.sync_copy(iy_h.at[wid], iv_y)
    copies = []
    for iv, rows, table in ((iv_t, rt, tt_h), (iv_a, ra, ta_h),
                            (iv_p, rp, tp_h), (iv_y, ry, ty_h)):
        for j in range(N_CHUNKS):
            copies.append(pltpu.async_copy(
                table.at[iv.at[j]], rows.at[pl.ds(j * CHUNK, CHUNK)], sem))
    for cp in copies:
        cp.wait()
    pltpu.sync_copy(rt, et_h.at[pl.ds(base, B_PER_W)])
    pltpu.sync_copy(ra, ea_h.at[pl.ds(base, B_PER_W)])
    pltpu.sync_copy(rp, ep_h.at[pl.ds(base, B_PER_W)])
    pltpu.sync_copy(ry, ey_h.at[pl.ds(base, B_PER_W)])


_gather = pl.kernel(
    _gather_body,
    out_type=[
        jax.ShapeDtypeStruct((B, D_TITLE), jnp.float32),
        jax.ShapeDtypeStruct((B, D_AUTHOR), jnp.float32),
        jax.ShapeDtypeStruct((B, D_PUB), jnp.float32),
        jax.ShapeDtypeStruct((B, D_YEAR_PAD), jnp.float32),
    ],
    mesh=plsc.VectorSubcoreMesh(core_axis_name="c", subcore_axis_name="s"),
    scratch_types=[
        pltpu.VMEM((N_CHUNKS, CHUNK), jnp.int32),
        pltpu.VMEM((N_CHUNKS, CHUNK), jnp.int32),
        pltpu.VMEM((N_CHUNKS, CHUNK), jnp.int32),
        pltpu.VMEM((N_CHUNKS, CHUNK), jnp.int32),
        pltpu.VMEM((B_PER_W, D_TITLE), jnp.float32),
        pltpu.VMEM((B_PER_W, D_AUTHOR), jnp.float32),
        pltpu.VMEM((B_PER_W, D_PUB), jnp.float32),
        pltpu.VMEM((B_PER_W, D_YEAR_PAD), jnp.float32),
        pltpu.SemaphoreType.DMA,
    ],
)


BM = 2048


def _mlp_body(et_ref, ea_ref, ep_ref, ey_ref, w1_ref, b1_ref, w2_ref, b2_ref,
              o_ref):
    h = jnp.dot(et_ref[...], w1_ref[0:32],
                preferred_element_type=jnp.float32)
    h += jnp.dot(ea_ref[...], w1_ref[32:64],
                 preferred_element_type=jnp.float32)
    h += jnp.dot(ep_ref[...], w1_ref[64:80],
                 preferred_element_type=jnp.float32)
    h += jnp.dot(ey_ref[...], w1_ref[80:96],
                 preferred_element_type=jnp.float32)
    h = jnp.maximum(h + b1_ref[...], 0.0)
    o_ref[...] = jnp.dot(h, w2_ref[...],
                         preferred_element_type=jnp.float32) + b2_ref[...]


def _mlp(et, ea, ep, ey, w1t, b1, w2t, b2):
    return pl.pallas_call(
        _mlp_body,
        grid=(B // BM,),
        in_specs=[
            pl.BlockSpec((BM, D_TITLE), lambda i: (i, 0)),
            pl.BlockSpec((BM, D_AUTHOR), lambda i: (i, 0)),
            pl.BlockSpec((BM, D_PUB), lambda i: (i, 0)),
            pl.BlockSpec((BM, D_YEAR_PAD), lambda i: (i, 0)),
            pl.BlockSpec((K_PAD, HIDDEN), lambda i: (0, 0)),
            pl.BlockSpec((1, HIDDEN), lambda i: (0, 0)),
            pl.BlockSpec((HIDDEN, EMBED_DIM), lambda i: (0, 0)),
            pl.BlockSpec((1, EMBED_DIM), lambda i: (0, 0)),
        ],
        out_specs=pl.BlockSpec((BM, EMBED_DIM), lambda i: (i, 0)),
        out_shape=jax.ShapeDtypeStruct((B, EMBED_DIM), jnp.float32),
    )(et, ea, ep, ey, w1t, b1, w2t, b2)


def kernel(book_title, book_author, book_publisher, book_year_of_publication,
           T_title, T_author, T_pub, T_year, W1, b1, W2, b2):
    it = book_title.astype(jnp.int32).reshape(NW, N_CHUNKS, CHUNK)
    ia = book_author.astype(jnp.int32).reshape(NW, N_CHUNKS, CHUNK)
    ip = book_publisher.astype(jnp.int32).reshape(NW, N_CHUNKS, CHUNK)
    iy = book_year_of_publication.astype(jnp.int32).reshape(NW, N_CHUNKS, CHUNK)
    ty16 = jnp.concatenate(
        [T_year, jnp.zeros((T_year.shape[0], D_YEAR_PAD - 8), T_year.dtype)],
        axis=1)
    et, ea, ep, ey = _gather(it, ia, ip, iy, T_title, T_author, T_pub, ty16)
    w1t = jnp.concatenate(
        [W1.T, jnp.zeros((K_PAD - W1.shape[1], HIDDEN), W1.dtype)], axis=0)
    return _mlp(et, ea, ep, ey, w1t, b1.reshape(1, HIDDEN), W2.T,
                b2.reshape(1, EMBED_DIM))


# speedup vs baseline: 1.8791x; 1.8791x over previous
import jax
import jax.numpy as jnp
from jax import lax
from jax.experimental import pallas as pl
from jax.experimental.pallas import tpu as pltpu
from jax.experimental.pallas import tpu_sc as plsc

B = 16384
D_TITLE, D_AUTHOR, D_PUB, D_YEAR_PAD = 32, 32, 16, 16
HIDDEN = 512
EMBED_DIM = 128
K_PAD = D_TITLE + D_AUTHOR + D_PUB + D_YEAR_PAD

NC, NS = 2, 16
NW = NC * NS
B_PER_W = B // NW
CHUNK = 128
N_CHUNKS = B_PER_W // CHUNK


def _gather_body(it_h, ia_h, ip_h, iy_h, tt_h, ta_h, tp_h, ty_h,
                 et_h, ea_h, ep_h, ey_h,
                 iv_t, iv_a, iv_p, iv_y, rt, ra, rp, ry, sem):
    wid = lax.axis_index("s") * NC + lax.axis_index("c")
    base = wid * B_PER_W
    pltpu.sync_copy(it_h.at[wid], iv_t)
    pltpu.sync_copy(ia_h.at[wid], iv_a)
    pltpu.sync_copy(ip_h.at[wid], iv_p)
    pltpu.sync_copy(iy_h.at[wid], iv_y)
    copies = []
    for iv, rows, table in ((iv_t, rt, tt_h), (iv_a, ra, ta_h),
                            (iv_p, rp, tp_h), (iv_y, ry, ty_h)):
        for j in range(N_CHUNKS):
            copies.append(pltpu.async_copy(
                table.at[iv.at[j]], rows.at[pl.ds(j * CHUNK, CHUNK)], sem))
    for cp in copies:
        cp.wait()
    pltpu.sync_copy(rt, et_h.at[pl.ds(base, B_PER_W)])
    pltpu.sync_copy(ra, ea_h.at[pl.ds(base, B_PER_W)])
    pltpu.sync_copy(rp, ep_h.at[pl.ds(base, B_PER_W)])
    pltpu.sync_copy(ry, ey_h.at[pl.ds(base, B_PER_W)])


_gather_cache = {}


def _get_gather():
    if "k" not in _gather_cache:
        _gather_cache["k"] = _make_gather()
    return _gather_cache["k"]


def _make_gather():
    return pl.kernel(
        _gather_body,
        out_type=[
            jax.ShapeDtypeStruct((B, D_TITLE), jnp.float32),
            jax.ShapeDtypeStruct((B, D_AUTHOR), jnp.float32),
            jax.ShapeDtypeStruct((B, D_PUB), jnp.float32),
            jax.ShapeDtypeStruct((B, D_YEAR_PAD), jnp.float32),
        ],
        mesh=plsc.VectorSubcoreMesh(core_axis_name="c", subcore_axis_name="s"),
        scratch_types=[
            pltpu.VMEM((N_CHUNKS, CHUNK), jnp.int32),
            pltpu.VMEM((N_CHUNKS, CHUNK), jnp.int32),
            pltpu.VMEM((N_CHUNKS, CHUNK), jnp.int32),
            pltpu.VMEM((N_CHUNKS, CHUNK), jnp.int32),
            pltpu.VMEM((B_PER_W, D_TITLE), jnp.float32),
            pltpu.VMEM((B_PER_W, D_AUTHOR), jnp.float32),
            pltpu.VMEM((B_PER_W, D_PUB), jnp.float32),
            pltpu.VMEM((B_PER_W, D_YEAR_PAD), jnp.float32),
            pltpu.SemaphoreType.DMA,
        ],
        compiler_params=pltpu.CompilerParams(use_tc_tiling_on_sc=False),
    )


BM = 2048


def _mlp_body(et_ref, ea_ref, ep_ref, ey_ref, w1_ref, b1_ref, w2_ref, b2_ref,
              o_ref):
    h = jnp.dot(et_ref[...], w1_ref[0:32],
                preferred_element_type=jnp.float32)
    h += jnp.dot(ea_ref[...], w1_ref[32:64],
                 preferred_element_type=jnp.float32)
    h += jnp.dot(ep_ref[...], w1_ref[64:80],
                 preferred_element_type=jnp.float32)
    h += jnp.dot(ey_ref[...], w1_ref[80:96],
                 preferred_element_type=jnp.float32)
    h = jnp.maximum(h + b1_ref[...], 0.0)
    o_ref[...] = jnp.dot(h, w2_ref[...],
                         preferred_element_type=jnp.float32) + b2_ref[...]


def _mlp(et, ea, ep, ey, w1t, b1, w2t, b2):
    return pl.pallas_call(
        _mlp_body,
        grid=(B // BM,),
        in_specs=[
            pl.BlockSpec((BM, D_TITLE), lambda i: (i, 0)),
            pl.BlockSpec((BM, D_AUTHOR), lambda i: (i, 0)),
            pl.BlockSpec((BM, D_PUB), lambda i: (i, 0)),
            pl.BlockSpec((BM, D_YEAR_PAD), lambda i: (i, 0)),
            pl.BlockSpec((K_PAD, HIDDEN), lambda i: (0, 0)),
            pl.BlockSpec((1, HIDDEN), lambda i: (0, 0)),
            pl.BlockSpec((HIDDEN, EMBED_DIM), lambda i: (0, 0)),
            pl.BlockSpec((1, EMBED_DIM), lambda i: (0, 0)),
        ],
        out_specs=pl.BlockSpec((BM, EMBED_DIM), lambda i: (i, 0)),
        out_shape=jax.ShapeDtypeStruct((B, EMBED_DIM), jnp.float32),
    )(et, ea, ep, ey, w1t, b1, w2t, b2)


def kernel(book_title, book_author, book_publisher, book_year_of_publication,
           T_title, T_author, T_pub, T_year, W1, b1, W2, b2):
    it = book_title.astype(jnp.int32).reshape(NW, N_CHUNKS, CHUNK)
    ia = book_author.astype(jnp.int32).reshape(NW, N_CHUNKS, CHUNK)
    ip = book_publisher.astype(jnp.int32).reshape(NW, N_CHUNKS, CHUNK)
    iy = book_year_of_publication.astype(jnp.int32).reshape(NW, N_CHUNKS, CHUNK)
    ty16 = jnp.concatenate(
        [T_year, jnp.zeros((T_year.shape[0], D_YEAR_PAD - 8), T_year.dtype)],
        axis=1)
    et, ea, ep, ey = _get_gather()(it, ia, ip, iy, T_title, T_author, T_pub,
                                   ty16)
    w1t = jnp.concatenate(
        [W1.T, jnp.zeros((K_PAD - W1.shape[1], HIDDEN), W1.dtype)], axis=0)
    return _mlp(et, ea, ep, ey, w1t, b1.reshape(1, HIDDEN), W2.T,
                b2.reshape(1, EMBED_DIM))


# speedup vs baseline: 2.1151x; 1.1256x over previous
import jax
import jax.numpy as jnp
from jax import lax
from jax.experimental import pallas as pl
from jax.experimental.pallas import tpu as pltpu
from jax.experimental.pallas import tpu_sc as plsc

B = 16384
SEG = 32
N_TABLES = 4
X_DIM = SEG * N_TABLES
HIDDEN = 512
EMBED_DIM = 128

NC, NS = 2, 16
NW = NC * NS
B_PER_W = B // NW
CHUNK = 128
N_CHUNKS = B_PER_W // CHUNK
N_IDX_ROWS = B // CHUNK


def _gather_body(it_h, ia_h, ip_h, iy_h, tt_h, ta_h, tp_h, ty_h, x_h,
                 iv, s0, s1, s2, s3, sem):
    stage = (s0, s1, s2, s3)
    wid = lax.axis_index("s") * NC + lax.axis_index("c")
    base = wid * B_PER_W
    row0 = wid * N_CHUNKS
    for k, idx_h in enumerate((it_h, ia_h, ip_h, iy_h)):
        pltpu.sync_copy(idx_h.at[pl.ds(row0, N_CHUNKS)],
                        iv.at[pl.ds(k * N_CHUNKS, N_CHUNKS)])
    copies = []
    for k, table in enumerate((tt_h, ta_h, tp_h, ty_h)):
        for j in range(N_CHUNKS):
            copies.append(pltpu.async_copy(
                table.at[iv.at[k * N_CHUNKS + j]],
                stage[k].at[pl.ds(j * CHUNK, CHUNK)],
                sem))
    for cp in copies:
        cp.wait()
    out_copies = [
        pltpu.async_copy(
            stage[k], x_h.at[pl.ds(base, B_PER_W), pl.ds(k * SEG, SEG)], sem)
        for k in range(N_TABLES)
    ]
    for cp in out_copies:
        cp.wait()


_gather_cache = {}


def _get_gather():
    if "k" not in _gather_cache:
        _gather_cache["k"] = pl.kernel(
            _gather_body,
            out_type=jax.ShapeDtypeStruct((B, X_DIM), jnp.float32),
            mesh=plsc.VectorSubcoreMesh(core_axis_name="c",
                                        subcore_axis_name="s"),
            scratch_types=[
                pltpu.VMEM((N_TABLES * N_CHUNKS, CHUNK), jnp.int32),
                pltpu.VMEM((B_PER_W, SEG), jnp.float32),
                pltpu.VMEM((B_PER_W, SEG), jnp.float32),
                pltpu.VMEM((B_PER_W, SEG), jnp.float32),
                pltpu.VMEM((B_PER_W, SEG), jnp.float32),
                pltpu.SemaphoreType.DMA,
            ],
            compiler_params=pltpu.CompilerParams(use_tc_tiling_on_sc=False),
        )
    return _gather_cache["k"]


BM = 2048


def _mlp_body(x_ref, w1_ref, b1_ref, w2_ref, b2_ref, o_ref):
    h = jnp.dot(x_ref[...], w1_ref[...], preferred_element_type=jnp.float32)
    h = jnp.maximum(h + b1_ref[...], 0.0)
    o_ref[...] = jnp.dot(h, w2_ref[...],
                         preferred_element_type=jnp.float32) + b2_ref[...]


def _mlp(x, w1t, b1, w2t, b2):
    return pl.pallas_call(
        _mlp_body,
        grid=(B // BM,),
        in_specs=[
            pl.BlockSpec((BM, X_DIM), lambda i: (i, 0)),
            pl.BlockSpec((X_DIM, HIDDEN), lambda i: (0, 0)),
            pl.BlockSpec((1, HIDDEN), lambda i: (0, 0)),
            pl.BlockSpec((HIDDEN, EMBED_DIM), lambda i: (0, 0)),
            pl.BlockSpec((1, EMBED_DIM), lambda i: (0, 0)),
        ],
        out_specs=pl.BlockSpec((BM, EMBED_DIM), lambda i: (i, 0)),
        out_shape=jax.ShapeDtypeStruct((B, EMBED_DIM), jnp.float32),
    )(x, w1t, b1, w2t, b2)


def kernel(book_title, book_author, book_publisher, book_year_of_publication,
           T_title, T_author, T_pub, T_year, W1, b1, W2, b2):
    it = book_title.astype(jnp.int32).reshape(N_IDX_ROWS, CHUNK)
    ia = book_author.astype(jnp.int32).reshape(N_IDX_ROWS, CHUNK)
    ip = book_publisher.astype(jnp.int32).reshape(N_IDX_ROWS, CHUNK)
    iy = book_year_of_publication.astype(jnp.int32).reshape(N_IDX_ROWS, CHUNK)
    tp32 = jnp.concatenate(
        [T_pub, jnp.zeros((T_pub.shape[0], SEG - T_pub.shape[1]),
                          T_pub.dtype)], axis=1)
    ty32 = jnp.concatenate(
        [T_year, jnp.zeros((T_year.shape[0], SEG - T_year.shape[1]),
                           T_year.dtype)], axis=1)
    x = _get_gather()(it, ia, ip, iy, T_title, T_author, tp32, ty32)
    w1t = W1.T
    w1p = jnp.zeros((X_DIM, HIDDEN), W1.dtype)
    w1p = w1p.at[0:64].set(w1t[0:64])
    w1p = w1p.at[64:80].set(w1t[64:80])
    w1p = w1p.at[96:104].set(w1t[80:88])
    return _mlp(x, w1p, b1.reshape(1, HIDDEN), W2.T, b2.reshape(1, EMBED_DIM))


# speedup vs baseline: 2.2982x; 1.0865x over previous
import jax
import jax.numpy as jnp
from jax import lax
from jax.experimental import pallas as pl
from jax.experimental.pallas import tpu as pltpu
from jax.experimental.pallas import tpu_sc as plsc

B = 16384
SEG = 32
N_TABLES = 4
X_DIM = SEG * N_TABLES
HIDDEN = 512
EMBED_DIM = 128

NC, NS = 2, 16
NW = NC * NS
B_PER_W = B // NW
CHUNK = 128
N_CHUNKS = B_PER_W // CHUNK
N_IDX_ROWS = B // CHUNK


def _gather_body(it_h, ia_h, ip_h, iy_h, tt_h, ta_h, tp_h, ty_h, x_h,
                 iv, s0, s1, s2, s3, sem):
    stage = (s0, s1, s2, s3)
    wid = lax.axis_index("s") * NC + lax.axis_index("c")
    base = wid * B_PER_W
    row0 = wid * N_CHUNKS
    for k, idx_h in enumerate((it_h, ia_h, ip_h, iy_h)):
        pltpu.sync_copy(idx_h.at[pl.ds(row0, N_CHUNKS)],
                        iv.at[pl.ds(k * N_CHUNKS, N_CHUNKS)])
    copies = []
    for k, table in enumerate((tt_h, ta_h, tp_h, ty_h)):
        for j in range(N_CHUNKS):
            copies.append(pltpu.async_copy(
                table.at[iv.at[k * N_CHUNKS + j]],
                stage[k].at[pl.ds(j * CHUNK, CHUNK)],
                sem))
    for cp in copies:
        cp.wait()
    out_copies = []
    for stg, col in ((s0, 0), (s1, 32), (s2, 64), (s3, 80),
                     (s2, 96), (s3, 112)):
        out_copies.append(pltpu.async_copy(
            stg, x_h.at[pl.ds(base, B_PER_W), pl.ds(col, stg.shape[1])],
            sem))
    for cp in out_copies:
        cp.wait()


_gather_cache = {}


def _get_gather():
    if "k" not in _gather_cache:
        _gather_cache["k"] = pl.kernel(
            _gather_body,
            out_type=jax.ShapeDtypeStruct((B, X_DIM), jnp.float32),
            mesh=plsc.VectorSubcoreMesh(core_axis_name="c",
                                        subcore_axis_name="s"),
            scratch_types=[
                pltpu.VMEM((N_TABLES * N_CHUNKS, CHUNK), jnp.int32),
                pltpu.VMEM((B_PER_W, 32), jnp.float32),
                pltpu.VMEM((B_PER_W, 32), jnp.float32),
                pltpu.VMEM((B_PER_W, 16), jnp.float32),
                pltpu.VMEM((B_PER_W, 16), jnp.float32),
                pltpu.SemaphoreType.DMA,
            ],
            compiler_params=pltpu.CompilerParams(use_tc_tiling_on_sc=False),
        )
    return _gather_cache["k"]


BM = 2048


def _mlp_body(x_ref, w1_ref, b1_ref, w2_ref, b2_ref, o_ref):
    h = jnp.dot(x_ref[...], w1_ref[...], preferred_element_type=jnp.float32)
    h = jnp.maximum(h + b1_ref[...], 0.0)
    o_ref[...] = jnp.dot(h, w2_ref[...],
                         preferred_element_type=jnp.float32) + b2_ref[...]


def _mlp(x, w1t, b1, w2t, b2):
    return pl.pallas_call(
        _mlp_body,
        grid=(B // BM,),
        in_specs=[
            pl.BlockSpec((BM, X_DIM), lambda i: (i, 0)),
            pl.BlockSpec((X_DIM, HIDDEN), lambda i: (0, 0)),
            pl.BlockSpec((1, HIDDEN), lambda i: (0, 0)),
            pl.BlockSpec((HIDDEN, EMBED_DIM), lambda i: (0, 0)),
            pl.BlockSpec((1, EMBED_DIM), lambda i: (0, 0)),
        ],
        out_specs=pl.BlockSpec((BM, EMBED_DIM), lambda i: (i, 0)),
        out_shape=jax.ShapeDtypeStruct((B, EMBED_DIM), jnp.float32),
    )(x, w1t, b1, w2t, b2)


def kernel(book_title, book_author, book_publisher, book_year_of_publication,
           T_title, T_author, T_pub, T_year, W1, b1, W2, b2):
    it = book_title.astype(jnp.int32).reshape(N_IDX_ROWS, CHUNK)
    ia = book_author.astype(jnp.int32).reshape(N_IDX_ROWS, CHUNK)
    ip = book_publisher.astype(jnp.int32).reshape(N_IDX_ROWS, CHUNK)
    iy = book_year_of_publication.astype(jnp.int32).reshape(N_IDX_ROWS, CHUNK)
    ty16 = jnp.concatenate(
        [T_year, jnp.zeros((T_year.shape[0], 8), T_year.dtype)], axis=1)
    x = _get_gather()(it, ia, ip, iy, T_title, T_author, T_pub, ty16)
    w1p = jnp.concatenate(
        [W1.T, jnp.zeros((X_DIM - W1.shape[1], HIDDEN), W1.dtype)], axis=0)
    return _mlp(x, w1p, b1.reshape(1, HIDDEN), W2.T, b2.reshape(1, EMBED_DIM))


# speedup vs baseline: 2.3071x; 1.0039x over previous
import functools

import jax
import jax.numpy as jnp
from jax import lax
from jax.experimental import pallas as pl
from jax.experimental.pallas import tpu as pltpu
from jax.experimental.pallas import tpu_sc as plsc

B = 16384
HIDDEN = 512
EMBED_DIM = 128
X_DIM = 128

VP_BIG, QB_BIG, GRID_BIG, D_BIG = 25088, 3584, 7, 32
VP_PUB, QB_PUB, GRID_PUB, D_PUB = 2560, 512, 5, 16
D_YEAR = 16

NC, NS = 2, 16
NW = NC * NS
B_PER_W = B // NW
CHUNK = 128
N_CHUNKS = B_PER_W // CHUNK
N_IDX_ROWS = B // CHUNK
L = 16


def _pack_body_big(i0, i1, i2, i3, o_ref):
    z = jnp.concatenate([i0[...], i1[...], i2[...], i3[...]], axis=0)
    o_ref[...] = z.T


def _pack_big(tt):
    specs = [
        pl.BlockSpec((D_BIG, QB_BIG),
                     functools.partial(lambda k, i: (0, GRID_BIG * k + i), k))
        for k in range(4)
    ]
    return pl.pallas_call(
        _pack_body_big,
        grid=(GRID_BIG,),
        in_specs=specs,
        out_specs=pl.BlockSpec((QB_BIG, 128), lambda i: (i, 0)),
        out_shape=jax.ShapeDtypeStruct((VP_BIG, 128), jnp.float32),
    )(tt, tt, tt, tt)


def _pack_body_pub(i0, i1, i2, i3, i4, i5, i6, i7, o_ref):
    z = jnp.concatenate(
        [r[...] for r in (i0, i1, i2, i3, i4, i5, i6, i7)], axis=0)
    o_ref[...] = z.T


def _pack_pub(tt):
    specs = [
        pl.BlockSpec((D_PUB, QB_PUB),
                     functools.partial(lambda k, i: (0, GRID_PUB * k + i), k))
        for k in range(8)
    ]
    return pl.pallas_call(
        _pack_body_pub,
        grid=(GRID_PUB,),
        in_specs=specs,
        out_specs=pl.BlockSpec((QB_PUB, 128), lambda i: (i, 0)),
        out_shape=jax.ShapeDtypeStruct((VP_PUB, 128), jnp.float32),
    )(*([tt] * 8))


def _extract(stage, si_ref, si_row, xstage, row0, col0, width):
    @pl.loop(0, CHUNK // L)
    def _(g):
        rows = jax.lax.iota(jnp.int32, L) + (g * L)
        si = plsc.load_gather(si_ref, [jnp.full((L,), si_row, jnp.int32),
                                       rows])
        colbase = si * width
        orows = rows + row0
        for c in range(width):
            v = plsc.load_gather(stage, [rows, colbase + c])
            plsc.store_scatter(xstage, [orows,
                                        jnp.full((L,), col0 + c, jnp.int32)],
                               v)


def _zero_cols(xstage, width):
    zeros = jnp.zeros((L,), jnp.float32)

    @pl.loop(0, B_PER_W // L)
    def _(g):
        orows = jax.lax.iota(jnp.int32, L) + (g * L)
        for c in range(width):
            plsc.store_scatter(xstage, [orows,
                                        jnp.full((L,), 96 + c, jnp.int32)],
                               zeros)


def _gather_body(qt_h, qa_h, qp_h, iy_h, st_h, sa_h, sp_h,
                 tt_h, ta_h, tp_h, ty_h, x_h,
                 iv, sv, g0, g1, ybuf, xstage, sem):
    gbuf = (g0, g1)
    wid = lax.axis_index("s") * NC + lax.axis_index("c")
    base = wid * B_PER_W
    row0 = wid * N_CHUNKS
    for k, idx_h in enumerate((qt_h, qa_h, qp_h, iy_h)):
        pltpu.sync_copy(idx_h.at[pl.ds(row0, N_CHUNKS)],
                        iv.at[pl.ds(k * N_CHUNKS, N_CHUNKS)])
    for k, s_h in enumerate((st_h, sa_h, sp_h)):
        pltpu.sync_copy(s_h.at[pl.ds(row0, N_CHUNKS)],
                        sv.at[pl.ds(k * N_CHUNKS, N_CHUNKS)])
    ycopies = [
        pltpu.async_copy(ty_h.at[iv.at[3 * N_CHUNKS + j]],
                         ybuf.at[pl.ds(j * CHUNK, CHUNK)], sem)
        for j in range(N_CHUNKS)
    ]
    jobs = []
    for j in range(N_CHUNKS):
        jobs.append((tt_h, 0 * N_CHUNKS + j, 0, 0, D_BIG))
    for j in range(N_CHUNKS):
        jobs.append((ta_h, 1 * N_CHUNKS + j, 1, 32, D_BIG))
    for j in range(N_CHUNKS):
        jobs.append((tp_h, 2 * N_CHUNKS + j, 2, 64, D_PUB))
    copies = [None, None]
    for n in range(2):
        table, ir, _, _, _ = jobs[n]
        copies[n] = pltpu.async_copy(table.at[iv.at[ir]], gbuf[n], sem)
    for n, (table, ir, sr, col0, width) in enumerate(jobs):
        copies[n % 2].wait()
        _extract(gbuf[n % 2], sv, sr * N_CHUNKS + (ir % N_CHUNKS), xstage,
                 (ir % N_CHUNKS) * CHUNK, col0, width)
        nxt = n + 2
        if nxt < len(jobs):
            t2, ir2, _, _, _ = jobs[nxt]
            copies[nxt % 2] = pltpu.async_copy(t2.at[iv.at[ir2]],
                                               gbuf[nxt % 2], sem)
    for cp in ycopies:
        cp.wait()
    @pl.loop(0, B_PER_W // L)
    def _(g):
        rows = jax.lax.iota(jnp.int32, L) + (g * L)
        for c in range(D_YEAR):
            v = plsc.load_gather(ybuf, [rows, jnp.full((L,), c, jnp.int32)])
            plsc.store_scatter(xstage, [rows,
                                        jnp.full((L,), 80 + c, jnp.int32)], v)
    _zero_cols(xstage, 32)
    pltpu.sync_copy(xstage, x_h.at[pl.ds(base, B_PER_W)])


_gather_cache = {}


def _get_gather():
    if "k" not in _gather_cache:
        _gather_cache["k"] = pl.kernel(
            _gather_body,
            out_type=jax.ShapeDtypeStruct((B, X_DIM), jnp.float32),
            mesh=plsc.VectorSubcoreMesh(core_axis_name="c",
                                        subcore_axis_name="s"),
            scratch_types=[
                pltpu.VMEM((4 * N_CHUNKS, CHUNK), jnp.int32),
                pltpu.VMEM((3 * N_CHUNKS, CHUNK), jnp.int32),
                pltpu.VMEM((CHUNK, 128), jnp.float32),
                pltpu.VMEM((CHUNK, 128), jnp.float32),
                pltpu.VMEM((B_PER_W, D_YEAR), jnp.float32),
                pltpu.VMEM((B_PER_W, X_DIM), jnp.float32),
                pltpu.SemaphoreType.DMA,
            ],
            compiler_params=pltpu.CompilerParams(use_tc_tiling_on_sc=False,
                                                 needs_layout_passes=False),
        )
    return _gather_cache["k"]


BM = 2048


def _mlp_body(x_ref, w1_ref, b1_ref, w2_ref, b2_ref, o_ref):
    h = jnp.dot(x_ref[...], w1_ref[...], preferred_element_type=jnp.float32)
    h = jnp.maximum(h + b1_ref[...], 0.0)
    o_ref[...] = jnp.dot(h, w2_ref[...],
                         preferred_element_type=jnp.float32) + b2_ref[...]


def _mlp(x, w1t, b1, w2t, b2):
    return pl.pallas_call(
        _mlp_body,
        grid=(B // BM,),
        in_specs=[
            pl.BlockSpec((BM, X_DIM), lambda i: (i, 0)),
            pl.BlockSpec((X_DIM, HIDDEN), lambda i: (0, 0)),
            pl.BlockSpec((1, HIDDEN), lambda i: (0, 0)),
            pl.BlockSpec((HIDDEN, EMBED_DIM), lambda i: (0, 0)),
            pl.BlockSpec((1, EMBED_DIM), lambda i: (0, 0)),
        ],
        out_specs=pl.BlockSpec((BM, EMBED_DIM), lambda i: (i, 0)),
        out_shape=jax.ShapeDtypeStruct((B, EMBED_DIM), jnp.float32),
    )(x, w1t, b1, w2t, b2)


def kernel(book_title, book_author, book_publisher, book_year_of_publication,
           T_title, T_author, T_pub, T_year, W1, b1, W2, b2):
    it = book_title.astype(jnp.int32)
    ia = book_author.astype(jnp.int32)
    ip = book_publisher.astype(jnp.int32)
    iy = book_year_of_publication.astype(jnp.int32)
    shp = (N_IDX_ROWS, CHUNK)
    qt = (it % VP_BIG).reshape(shp)
    st = (it // VP_BIG).reshape(shp)
    qa = (ia % VP_BIG).reshape(shp)
    sa = (ia // VP_BIG).reshape(shp)
    qp = (ip % VP_PUB).reshape(shp)
    sp = (ip // VP_PUB).reshape(shp)
    iyr = iy.reshape(shp)
    ttq = _pack_big(T_title.T)
    taq = _pack_big(T_author.T)
    tpq = _pack_pub(T_pub.T)
    ty16 = jnp.concatenate(
        [T_year, jnp.zeros((T_year.shape[0], 8), T_year.dtype)], axis=1)
    x = _get_gather()(qt, qa, qp, iyr, st, sa, sp, ttq, taq, tpq, ty16)
    w1p = jnp.concatenate(
        [W1.T, jnp.zeros((X_DIM - W1.shape[1], HIDDEN), W1.dtype)], axis=0)
    return _mlp(x, w1p, b1.reshape(1, HIDDEN), W2.T, b2.reshape(1, EMBED_DIM))
